# baseline (speedup 1.0000x reference)
import math

import jax
import jax.numpy as jnp
from jax.experimental import pallas as pl
from jax.experimental.pallas import tpu as pltpu

_C = math.sqrt(2.0 / math.pi)
_ROWS_PER_BLOCK = 1024


def _gelu_block(x_ref, o_ref):
    x = x_ref[...]
    inner = _C * (x + 0.044715 * (x * x * x))
    o_ref[...] = 0.5 * x * (1.0 + jnp.tanh(inner))


def kernel(x):
    B, T, D = x.shape
    rows = B * T
    x2 = x.reshape(rows, D)
    grid = rows // _ROWS_PER_BLOCK
    y2 = pl.pallas_call(
        _gelu_block,
        grid=(grid,),
        in_specs=[pl.BlockSpec((_ROWS_PER_BLOCK, D), lambda i: (i, 0))],
        out_specs=pl.BlockSpec((_ROWS_PER_BLOCK, D), lambda i: (i, 0)),
        out_shape=jax.ShapeDtypeStruct((rows, D), x.dtype),
        compiler_params=pltpu.CompilerParams(
            dimension_semantics=("arbitrary",),
        ),
    )(x2)
    return y2.reshape(B, T, D)


# speedup vs baseline: 1.0242x; 1.0242x over previous
import math

import jax
import jax.numpy as jnp
from jax.experimental import pallas as pl
from jax.experimental.pallas import tpu as pltpu

_C = math.sqrt(2.0 / math.pi)
_ROWS_PER_BLOCK = 2048


def _gelu_block(x_ref, o_ref):
    x = x_ref[...]
    inner = _C * (x + 0.044715 * (x * x * x))
    o_ref[...] = 0.5 * x * (1.0 + jnp.tanh(inner))


def kernel(x):
    B, T, D = x.shape
    rows = B * T
    x2 = x.reshape(rows, D)
    grid = rows // _ROWS_PER_BLOCK
    y2 = pl.pallas_call(
        _gelu_block,
        grid=(grid,),
        in_specs=[pl.BlockSpec((_ROWS_PER_BLOCK, D), lambda i: (i, 0))],
        out_specs=pl.BlockSpec((_ROWS_PER_BLOCK, D), lambda i: (i, 0)),
        out_shape=jax.ShapeDtypeStruct((rows, D), x.dtype),
        compiler_params=pltpu.CompilerParams(
            dimension_semantics=("arbitrary",),
        ),
    )(x2)
    return y2.reshape(B, T, D)


# speedup vs baseline: 1.0389x; 1.0144x over previous
import math

import jax
import jax.numpy as jnp
from jax.experimental import pallas as pl
from jax.experimental.pallas import tpu as pltpu

_C = math.sqrt(2.0 / math.pi)
_ROWS_PER_BLOCK = 3584


def _gelu_block(x_ref, o_ref):
    x = x_ref[...]
    inner = _C * (x + 0.044715 * (x * x * x))
    o_ref[...] = 0.5 * x * (1.0 + jnp.tanh(inner))


def kernel(x):
    B, T, D = x.shape
    rows = B * T
    x2 = x.reshape(rows, D)
    grid = pl.cdiv(rows, _ROWS_PER_BLOCK)
    y2 = pl.pallas_call(
        _gelu_block,
        grid=(grid,),
        in_specs=[pl.BlockSpec((_ROWS_PER_BLOCK, D), lambda i: (i, 0))],
        out_specs=pl.BlockSpec((_ROWS_PER_BLOCK, D), lambda i: (i, 0)),
        out_shape=jax.ShapeDtypeStruct((rows, D), x.dtype),
        compiler_params=pltpu.CompilerParams(
            dimension_semantics=("arbitrary",),
            vmem_limit_bytes=100 * 1024 * 1024,
        ),
    )(x2)
    return y2.reshape(B, T, D)
